# speedup vs baseline: 3.5540x; 3.5540x over previous
import functools

import jax
import jax.numpy as jnp
from jax.experimental import pallas as pl
from jax.experimental.pallas import tpu as pltpu

N = 10000
E = 320000
D = 128
ROWS = 1000


def _pre_body(x_ref, w1_ref, b1_ref, W_ref, dinv_ref, hh_ref):
    xb = x_ref[...]
    mean = jnp.mean(xb, axis=1, keepdims=True)
    cen = xb - mean
    var = jnp.mean(cen * cen, axis=1, keepdims=True)
    h1 = cen * jax.lax.rsqrt(var + 1e-5) * w1_ref[...] + b1_ref[...]
    h = jnp.dot(h1, W_ref[...], preferred_element_type=jnp.float32)
    hh_ref[...] = h * dinv_ref[...]


def _tc_pre(x, ln1_w, ln1_b, gcn_W, dinv_col):
    grid = (N // ROWS,)
    return pl.pallas_call(
        _pre_body,
        grid=grid,
        in_specs=[
            pl.BlockSpec((ROWS, D), lambda i: (i, 0)),
            pl.BlockSpec((D,), lambda i: (0,)),
            pl.BlockSpec((D,), lambda i: (0,)),
            pl.BlockSpec((D, D), lambda i: (0, 0)),
            pl.BlockSpec((ROWS, 1), lambda i: (i, 0)),
        ],
        out_specs=pl.BlockSpec((ROWS, D), lambda i: (i, 0)),
        out_shape=jax.ShapeDtypeStruct((N, D), jnp.float32),
    )(x, ln1_w, ln1_b, gcn_W, dinv_col)


def _post_body(x_ref, hh_ref, a0_ref, a1_ref, dinv_ref, gb_ref, s1_ref,
               w2_ref, b2_ref, ffW_ref, ffb_ref, s2_ref, out_ref):
    agg = a0_ref[...] + a1_ref[...] + hh_ref[...]
    m = agg * dinv_ref[...] + gb_ref[...]
    m = m * jax.nn.sigmoid(m)
    x1 = x_ref[...] + s1_ref[0, 0] * m
    mean = jnp.mean(x1, axis=1, keepdims=True)
    cen = x1 - mean
    var = jnp.mean(cen * cen, axis=1, keepdims=True)
    h2 = cen * jax.lax.rsqrt(var + 1e-5) * w2_ref[...] + b2_ref[...]
    f = jnp.dot(h2, ffW_ref[...], preferred_element_type=jnp.float32) + ffb_ref[...]
    f = f * jax.nn.sigmoid(f)
    out_ref[...] = x1 + s2_ref[0, 0] * f


def _tc_post(x, hh, agg0, agg1, dinv_col, gcn_b, alpha1, ln2_w, ln2_b,
             ff_W, ff_b, alpha2):
    grid = (N // ROWS,)
    row = lambda i: (i, 0)
    vec = lambda i: (0,)
    smem = pl.BlockSpec(memory_space=pltpu.SMEM)
    return pl.pallas_call(
        _post_body,
        grid=grid,
        in_specs=[
            pl.BlockSpec((ROWS, D), row),
            pl.BlockSpec((ROWS, D), row),
            pl.BlockSpec((ROWS, D), row),
            pl.BlockSpec((ROWS, D), row),
            pl.BlockSpec((ROWS, 1), row),
            pl.BlockSpec((D,), vec),
            smem,
            pl.BlockSpec((D,), vec),
            pl.BlockSpec((D,), vec),
            pl.BlockSpec((D, D), lambda i: (0, 0)),
            pl.BlockSpec((D,), vec),
            smem,
        ],
        out_specs=pl.BlockSpec((ROWS, D), row),
        out_shape=jax.ShapeDtypeStruct((N, D), jnp.float32),
    )(x, hh, agg0, agg1, dinv_col, gcn_b, alpha1.reshape(1, 1), ln2_w,
      ln2_b, ff_W, ff_b, alpha2.reshape(1, 1))


def kernel(x, edge_index, ln1_w, ln1_b, gcn_W, gcn_b, alpha1, alpha2,
           ln2_w, ln2_b, ff_W, ff_b):
    src = edge_index[0]
    dst = edge_index[1]
    deg = jnp.zeros((N,), jnp.float32).at[dst].add(1.0) + 1.0
    dinv_col = jax.lax.rsqrt(deg).reshape(N, 1)
    hh = _tc_pre(x, ln1_w, ln1_b, gcn_W, dinv_col)
    agg = jnp.zeros((N, D), jnp.float32).at[dst].add(hh[src])
    zeros = jnp.zeros((N, D), jnp.float32)
    return _tc_post(x, hh, agg, zeros, dinv_col, gcn_b, alpha1, ln2_w,
                    ln2_b, ff_W, ff_b, alpha2)


# speedup vs baseline: 17.8480x; 5.0219x over previous
import functools

import jax
import jax.numpy as jnp
from jax import lax
from jax.experimental import pallas as pl
from jax.experimental.pallas import tpu as pltpu
from jax.experimental.pallas import tpu_sc as plsc

N = 10000
E = 320000
D = 128
ROWS = 1000

NC = 2
NS = 16
NW = NC * NS
NP = 10240
EPW = E // NW
DEG_CHUNK = 2000


def _deg_body(dst_hbm, out_hbm, dstv, hist):
    wid = lax.axis_index("s") * NC + lax.axis_index("c")
    zeros = jnp.zeros((16,), jnp.float32)
    ones = jnp.ones((16,), jnp.float32)

    def zero_body(i, _):
        hist[pl.ds(i * 16, 16)] = zeros
        return 0

    lax.fori_loop(0, NP // 16, zero_body, 0)

    base = wid * EPW
    for c in range(EPW // DEG_CHUNK):
        pltpu.sync_copy(dst_hbm.at[pl.ds(base + c * DEG_CHUNK, DEG_CHUNK)], dstv)

        def scat_body(j, _):
            idx = dstv[pl.ds(j * 16, 16)]
            plsc.addupdate_scatter(hist, [idx], ones)
            return 0

        lax.fori_loop(0, DEG_CHUNK // 16, scat_body, 0)
    pltpu.sync_copy(hist, out_hbm.at[wid])


_deg_kernel = functools.partial(
    pl.kernel,
    out_type=jax.ShapeDtypeStruct((NW, NP), jnp.float32),
    mesh=plsc.VectorSubcoreMesh(core_axis_name="c", subcore_axis_name="s"),
    scratch_types=[
        pltpu.VMEM((DEG_CHUNK,), jnp.int32),
        pltpu.VMEM((NP,), jnp.float32),
    ],
    compiler_params=pltpu.CompilerParams(needs_layout_passes=False),
)(_deg_body)


EK = 80
ECHUNKS = EPW // EK
ZROWS = 128
SPN = NP // NS


def _edge_body(hh_hbm, src_hbm, dst_hbm, out_hbm, srcv, dstv, rows, zbuf,
               agg, sem):
    cid = lax.axis_index("c")
    sid = lax.axis_index("s")
    wid = sid * NC + cid
    zeros = jnp.zeros((16,), jnp.float32)

    def zb_body(i, _):
        for j in range(D // 16):
            zbuf[i, pl.ds(j * 16, 16)] = zeros
        return 0

    lax.fori_loop(0, ZROWS, zb_body, 0)
    r0 = sid * SPN
    for t in range(SPN // ZROWS):
        pltpu.sync_copy(zbuf, agg.at[pl.ds(r0 + t * ZROWS, ZROWS)])
    plsc.subcore_barrier()

    ebase = wid * EPW

    def chunk_body(c, _):
        off = ebase + c * EK
        pltpu.sync_copy(src_hbm.at[pl.ds(off, EK)], srcv)
        pltpu.sync_copy(dst_hbm.at[pl.ds(off, EK)], dstv)
        pltpu.async_copy(hh_hbm.at[srcv], rows, sem).wait()
        pltpu.sync_copy(rows, agg.at[dstv], add=True)
        return 0

    lax.fori_loop(0, ECHUNKS, chunk_body, 0)
    plsc.subcore_barrier()
    for t in range(SPN // ZROWS):
        pltpu.sync_copy(agg.at[pl.ds(r0 + t * ZROWS, ZROWS)],
                        out_hbm.at[cid, pl.ds(r0 + t * ZROWS, ZROWS)])


_edge_kernel = functools.partial(
    pl.kernel,
    out_type=jax.ShapeDtypeStruct((NC, NP, D), jnp.float32),
    mesh=plsc.VectorSubcoreMesh(core_axis_name="c", subcore_axis_name="s"),
    scratch_types=[
        pltpu.VMEM((EK,), jnp.int32),
        pltpu.VMEM((EK,), jnp.int32),
        pltpu.VMEM((EK, D), jnp.float32),
        pltpu.VMEM((ZROWS, D), jnp.float32),
        pltpu.VMEM_SHARED((NP, D), jnp.float32),
        pltpu.SemaphoreType.DMA,
    ],
    compiler_params=pltpu.CompilerParams(needs_layout_passes=False),
)(_edge_body)


def _pre_body(x_ref, w1_ref, b1_ref, W_ref, dinv_ref, hh_ref):
    xb = x_ref[...]
    mean = jnp.mean(xb, axis=1, keepdims=True)
    cen = xb - mean
    var = jnp.mean(cen * cen, axis=1, keepdims=True)
    h1 = cen * jax.lax.rsqrt(var + 1e-5) * w1_ref[...] + b1_ref[...]
    h = jnp.dot(h1, W_ref[...], preferred_element_type=jnp.float32)
    hh_ref[...] = h * dinv_ref[...]


def _tc_pre(x, ln1_w, ln1_b, gcn_W, dinv_col):
    grid = (N // ROWS,)
    return pl.pallas_call(
        _pre_body,
        grid=grid,
        in_specs=[
            pl.BlockSpec((ROWS, D), lambda i: (i, 0)),
            pl.BlockSpec((D,), lambda i: (0,)),
            pl.BlockSpec((D,), lambda i: (0,)),
            pl.BlockSpec((D, D), lambda i: (0, 0)),
            pl.BlockSpec((ROWS, 1), lambda i: (i, 0)),
        ],
        out_specs=pl.BlockSpec((ROWS, D), lambda i: (i, 0)),
        out_shape=jax.ShapeDtypeStruct((N, D), jnp.float32),
    )(x, ln1_w, ln1_b, gcn_W, dinv_col)


def _post_body(x_ref, hh_ref, a0_ref, a1_ref, dinv_ref, gb_ref, s1_ref,
               w2_ref, b2_ref, ffW_ref, ffb_ref, s2_ref, out_ref):
    agg = a0_ref[...] + a1_ref[...] + hh_ref[...]
    m = agg * dinv_ref[...] + gb_ref[...]
    m = m * jax.nn.sigmoid(m)
    x1 = x_ref[...] + s1_ref[0, 0] * m
    mean = jnp.mean(x1, axis=1, keepdims=True)
    cen = x1 - mean
    var = jnp.mean(cen * cen, axis=1, keepdims=True)
    h2 = cen * jax.lax.rsqrt(var + 1e-5) * w2_ref[...] + b2_ref[...]
    f = jnp.dot(h2, ffW_ref[...], preferred_element_type=jnp.float32) + ffb_ref[...]
    f = f * jax.nn.sigmoid(f)
    out_ref[...] = x1 + s2_ref[0, 0] * f


def _tc_post(x, hh, agg0, agg1, dinv_col, gcn_b, alpha1, ln2_w, ln2_b,
             ff_W, ff_b, alpha2):
    grid = (N // ROWS,)
    row = lambda i: (i, 0)
    vec = lambda i: (0,)
    smem = pl.BlockSpec(memory_space=pltpu.SMEM)
    return pl.pallas_call(
        _post_body,
        grid=grid,
        in_specs=[
            pl.BlockSpec((ROWS, D), row),
            pl.BlockSpec((ROWS, D), row),
            pl.BlockSpec((ROWS, D), row),
            pl.BlockSpec((ROWS, D), row),
            pl.BlockSpec((ROWS, 1), row),
            pl.BlockSpec((D,), vec),
            smem,
            pl.BlockSpec((D,), vec),
            pl.BlockSpec((D,), vec),
            pl.BlockSpec((D, D), lambda i: (0, 0)),
            pl.BlockSpec((D,), vec),
            smem,
        ],
        out_specs=pl.BlockSpec((ROWS, D), row),
        out_shape=jax.ShapeDtypeStruct((N, D), jnp.float32),
    )(x, hh, agg0, agg1, dinv_col, gcn_b, alpha1.reshape(1, 1), ln2_w,
      ln2_b, ff_W, ff_b, alpha2.reshape(1, 1))


def kernel(x, edge_index, ln1_w, ln1_b, gcn_W, gcn_b, alpha1, alpha2,
           ln2_w, ln2_b, ff_W, ff_b):
    src = edge_index[0]
    dst = edge_index[1]
    deg_parts = _deg_kernel(dst)
    deg = deg_parts.sum(axis=0)[:N] + 1.0
    dinv_col = jax.lax.rsqrt(deg).reshape(N, 1)
    hh = _tc_pre(x, ln1_w, ln1_b, gcn_W, dinv_col)
    aggs = _edge_kernel(hh, src, dst)
    return _tc_post(x, hh, aggs[0, :N], aggs[1, :N], dinv_col, gcn_b, alpha1, ln2_w,
                    ln2_b, ff_W, ff_b, alpha2)


# speedup vs baseline: 29.2562x; 1.6392x over previous
import functools

import jax
import jax.numpy as jnp
from jax import lax
from jax.experimental import pallas as pl
from jax.experimental.pallas import tpu as pltpu
from jax.experimental.pallas import tpu_sc as plsc

N = 10000
E = 320000
D = 128
ROWS = 1000

NC = 2
NS = 16
NW = NC * NS
NP = 10240
EPW = E // NW
DEG_CHUNK = 2000


def _deg_body(dst_hbm, out_hbm, dstv, hist):
    wid = lax.axis_index("s") * NC + lax.axis_index("c")
    zeros = jnp.zeros((16,), jnp.float32)
    ones = jnp.ones((16,), jnp.float32)

    def zero_body(i, _):
        hist[pl.ds(i * 16, 16)] = zeros
        return 0

    lax.fori_loop(0, NP // 16, zero_body, 0)

    base = wid * EPW
    for c in range(EPW // DEG_CHUNK):
        pltpu.sync_copy(dst_hbm.at[pl.ds(base + c * DEG_CHUNK, DEG_CHUNK)], dstv)

        def scat_body(j, _):
            idx = dstv[pl.ds(j * 16, 16)]
            plsc.addupdate_scatter(hist, [idx], ones)
            return 0

        lax.fori_loop(0, DEG_CHUNK // 16, scat_body, 0)
    pltpu.sync_copy(hist, out_hbm.at[wid])


_deg_kernel = functools.partial(
    pl.kernel,
    out_type=jax.ShapeDtypeStruct((NW, NP), jnp.float32),
    mesh=plsc.VectorSubcoreMesh(core_axis_name="c", subcore_axis_name="s"),
    scratch_types=[
        pltpu.VMEM((DEG_CHUNK,), jnp.int32),
        pltpu.VMEM((NP,), jnp.float32),
    ],
    compiler_params=pltpu.CompilerParams(needs_layout_passes=False),
)(_deg_body)


EK = 80
ECHUNKS = EPW // EK
ZROWS = 128
SPN = NP // NS


def _edge_body(hh_hbm, src_hbm, dst_hbm, out_hbm, sidx0, sidx1, didx0,
               didx1, rows0, rows1, agg, semg0, semg1, semi0, semi1):
    cid = lax.axis_index("c")
    sid = lax.axis_index("s")
    wid = sid * NC + cid
    zeros = jnp.zeros((16,), jnp.float32)
    sidx = (sidx0, sidx1)
    didx = (didx0, didx1)
    rows = (rows0, rows1)
    semg = (semg0, semg1)
    semi = (semi0, semi1)

    def zb_body(i, _):
        for j in range(D // 16):
            rows0[i, pl.ds(j * 16, 16)] = zeros
        return 0

    lax.fori_loop(0, EK, zb_body, 0)
    r0 = sid * SPN
    for t in range(SPN // EK):
        pltpu.sync_copy(rows0, agg.at[pl.ds(r0 + t * EK, EK)])
    plsc.subcore_barrier()

    ebase = wid * EPW
    pltpu.sync_copy(src_hbm.at[pl.ds(ebase, EK)], sidx0)
    pltpu.sync_copy(dst_hbm.at[pl.ds(ebase, EK)], didx0)
    pltpu.async_copy(hh_hbm.at[sidx0], rows0, semg0)
    pltpu.async_copy(src_hbm.at[pl.ds(ebase + EK, EK)], sidx1, semi1)
    pltpu.async_copy(dst_hbm.at[pl.ds(ebase + EK, EK)], didx1, semi1)

    def chunk_body(c, _):
        def step(a, b):
            pltpu.make_async_copy(hh_hbm.at[sidx[a]], rows[a], semg[a]).wait()

            @pl.when(c + 1 < ECHUNKS)
            def _():
                pltpu.make_async_copy(src_hbm.at[pl.ds(ebase, EK)], sidx[b],
                                      semi[b]).wait()
                pltpu.make_async_copy(dst_hbm.at[pl.ds(ebase, EK)], didx[b],
                                      semi[b]).wait()
                pltpu.async_copy(hh_hbm.at[sidx[b]], rows[b], semg[b])

            pltpu.sync_copy(rows[a], agg.at[didx[a]], add=True)

            @pl.when(c + 2 < ECHUNKS)
            def _():
                off = ebase + (c + 2) * EK
                pltpu.async_copy(src_hbm.at[pl.ds(off, EK)], sidx[a], semi[a])
                pltpu.async_copy(dst_hbm.at[pl.ds(off, EK)], didx[a], semi[a])

        @pl.when(c % 2 == 0)
        def _():
            step(0, 1)

        @pl.when(c % 2 == 1)
        def _():
            step(1, 0)

        return 0

    lax.fori_loop(0, ECHUNKS, chunk_body, 0)
    plsc.subcore_barrier()
    for t in range(SPN // EK):
        pltpu.sync_copy(agg.at[pl.ds(r0 + t * EK, EK)],
                        out_hbm.at[cid, pl.ds(r0 + t * EK, EK)])


_edge_kernel = functools.partial(
    pl.kernel,
    out_type=jax.ShapeDtypeStruct((NC, NP, D), jnp.float32),
    mesh=plsc.VectorSubcoreMesh(core_axis_name="c", subcore_axis_name="s"),
    scratch_types=[
        pltpu.VMEM((EK,), jnp.int32),
        pltpu.VMEM((EK,), jnp.int32),
        pltpu.VMEM((EK,), jnp.int32),
        pltpu.VMEM((EK,), jnp.int32),
        pltpu.VMEM((EK, D), jnp.float32),
        pltpu.VMEM((EK, D), jnp.float32),
        pltpu.VMEM_SHARED((NP, D), jnp.float32),
        pltpu.SemaphoreType.DMA,
        pltpu.SemaphoreType.DMA,
        pltpu.SemaphoreType.DMA,
        pltpu.SemaphoreType.DMA,
    ],
    compiler_params=pltpu.CompilerParams(needs_layout_passes=False),
)(_edge_body)


def _pre_body(x_ref, w1_ref, b1_ref, W_ref, dinv_ref, hh_ref):
    xb = x_ref[...]
    mean = jnp.mean(xb, axis=1, keepdims=True)
    cen = xb - mean
    var = jnp.mean(cen * cen, axis=1, keepdims=True)
    h1 = cen * jax.lax.rsqrt(var + 1e-5) * w1_ref[...] + b1_ref[...]
    h = jnp.dot(h1, W_ref[...], preferred_element_type=jnp.float32)
    hh_ref[...] = h * dinv_ref[...]


def _tc_pre(x, ln1_w, ln1_b, gcn_W, dinv_col):
    grid = (N // ROWS,)
    return pl.pallas_call(
        _pre_body,
        grid=grid,
        in_specs=[
            pl.BlockSpec((ROWS, D), lambda i: (i, 0)),
            pl.BlockSpec((D,), lambda i: (0,)),
            pl.BlockSpec((D,), lambda i: (0,)),
            pl.BlockSpec((D, D), lambda i: (0, 0)),
            pl.BlockSpec((ROWS, 1), lambda i: (i, 0)),
        ],
        out_specs=pl.BlockSpec((ROWS, D), lambda i: (i, 0)),
        out_shape=jax.ShapeDtypeStruct((N, D), jnp.float32),
    )(x, ln1_w, ln1_b, gcn_W, dinv_col)


def _post_body(x_ref, hh_ref, a0_ref, a1_ref, dinv_ref, gb_ref, s1_ref,
               w2_ref, b2_ref, ffW_ref, ffb_ref, s2_ref, out_ref):
    agg = a0_ref[...] + a1_ref[...] + hh_ref[...]
    m = agg * dinv_ref[...] + gb_ref[...]
    m = m * jax.nn.sigmoid(m)
    x1 = x_ref[...] + s1_ref[0, 0] * m
    mean = jnp.mean(x1, axis=1, keepdims=True)
    cen = x1 - mean
    var = jnp.mean(cen * cen, axis=1, keepdims=True)
    h2 = cen * jax.lax.rsqrt(var + 1e-5) * w2_ref[...] + b2_ref[...]
    f = jnp.dot(h2, ffW_ref[...], preferred_element_type=jnp.float32) + ffb_ref[...]
    f = f * jax.nn.sigmoid(f)
    out_ref[...] = x1 + s2_ref[0, 0] * f


def _tc_post(x, hh, agg0, agg1, dinv_col, gcn_b, alpha1, ln2_w, ln2_b,
             ff_W, ff_b, alpha2):
    grid = (N // ROWS,)
    row = lambda i: (i, 0)
    vec = lambda i: (0,)
    smem = pl.BlockSpec(memory_space=pltpu.SMEM)
    return pl.pallas_call(
        _post_body,
        grid=grid,
        in_specs=[
            pl.BlockSpec((ROWS, D), row),
            pl.BlockSpec((ROWS, D), row),
            pl.BlockSpec((ROWS, D), row),
            pl.BlockSpec((ROWS, D), row),
            pl.BlockSpec((ROWS, 1), row),
            pl.BlockSpec((D,), vec),
            smem,
            pl.BlockSpec((D,), vec),
            pl.BlockSpec((D,), vec),
            pl.BlockSpec((D, D), lambda i: (0, 0)),
            pl.BlockSpec((D,), vec),
            smem,
        ],
        out_specs=pl.BlockSpec((ROWS, D), row),
        out_shape=jax.ShapeDtypeStruct((N, D), jnp.float32),
    )(x, hh, agg0, agg1, dinv_col, gcn_b, alpha1.reshape(1, 1), ln2_w,
      ln2_b, ff_W, ff_b, alpha2.reshape(1, 1))


def kernel(x, edge_index, ln1_w, ln1_b, gcn_W, gcn_b, alpha1, alpha2,
           ln2_w, ln2_b, ff_W, ff_b):
    src = edge_index[0]
    dst = edge_index[1]
    deg_parts = _deg_kernel(dst)
    deg = deg_parts.sum(axis=0)[:N] + 1.0
    dinv_col = jax.lax.rsqrt(deg).reshape(N, 1)
    hh = _tc_pre(x, ln1_w, ln1_b, gcn_W, dinv_col)
    aggs = _edge_kernel(hh, src, dst)
    return _tc_post(x, hh, aggs[0, :N], aggs[1, :N], dinv_col, gcn_b, alpha1, ln2_w,
                    ln2_b, ff_W, ff_b, alpha2)
